# Initial kernel scaffold; baseline (speedup 1.0000x reference)
import jax
import jax.numpy as jnp
from jax.experimental import pallas as pl


def kernel(inputs, embeddings):
    raise NotImplementedError("write your pallas kernel here")


# speedup vs baseline: 1.7830x; 1.7830x over previous
import jax
import jax.numpy as jnp
from jax.experimental import pallas as pl
from jax.experimental.pallas import tpu as pltpu

LATENT = 64
CODES = 1024
BM = 1024


def _vq_block(x_ref, e_ref, o_ref):
    x = x_ref[...]
    e = e_ref[...]
    sim = jnp.dot(x, e, preferred_element_type=jnp.float32)
    e2 = jnp.sum(e * e, axis=0, keepdims=True)
    dist = e2 - 2.0 * sim
    minval = jnp.min(dist, axis=1, keepdims=True)
    iota = jax.lax.broadcasted_iota(jnp.int32, dist.shape, 1)
    idx = jnp.min(jnp.where(dist == minval, iota, CODES), axis=1, keepdims=True)
    onehot = (iota == idx).astype(jnp.float32)
    o_ref[...] = jax.lax.dot_general(
        onehot, e, (((1,), (1,)), ((), ())),
        preferred_element_type=jnp.float32)


def kernel(inputs, embeddings):
    shape = inputs.shape
    flat = inputs.reshape(-1, LATENT)
    n = flat.shape[0]
    out = pl.pallas_call(
        _vq_block,
        grid=(n // BM,),
        in_specs=[
            pl.BlockSpec((BM, LATENT), lambda i: (i, 0)),
            pl.BlockSpec((LATENT, CODES), lambda i: (0, 0)),
        ],
        out_specs=pl.BlockSpec((BM, LATENT), lambda i: (i, 0)),
        out_shape=jax.ShapeDtypeStruct((n, LATENT), jnp.float32),
    )(flat, embeddings)
    return out.reshape(shape)


# speedup vs baseline: 1.9912x; 1.1167x over previous
import functools
import jax
import jax.numpy as jnp
from jax import lax
from jax.experimental import pallas as pl
from jax.experimental.pallas import tpu as pltpu
from jax.experimental.pallas import tpu_sc as plsc

LATENT = 64
CODES = 1024
BM = 1024
NW = 32


CHUNK = 128


def _argmin_block(x_ref, e_ref, idx_ref):
    x = x_ref[...]
    e = e_ref[...]
    sim2 = jnp.dot(x, e + e, preferred_element_type=jnp.float32)
    e2 = jnp.sum(e * e, axis=0, keepdims=True)
    x2 = jnp.sum(x * x, axis=1, keepdims=True)
    bestv = (x2 + e2[:, :CHUNK]) - sim2[:, :CHUNK]
    besti0 = lax.broadcasted_iota(jnp.int32, (BM, CHUNK), 1).astype(jnp.float32)
    besti = besti0
    for c in range(1, CODES // CHUNK):
        sl = slice(c * CHUNK, (c + 1) * CHUNK)
        v = (x2 + e2[:, sl]) - sim2[:, sl]
        lt = v < bestv
        bestv = jnp.where(lt, v, bestv)
        besti = jnp.where(lt, besti0 + float(c * CHUNK), besti)
    minval = jnp.min(bestv, axis=1, keepdims=True)
    idxv = jnp.min(jnp.where(bestv == minval, besti, float(CODES)), axis=1)
    idx_ref[...] = idxv.astype(jnp.int32).reshape(8, BM // 8)


def _tc_indices(flat, embeddings):
    n = flat.shape[0]
    nb = n // BM
    idx = pl.pallas_call(
        _argmin_block,
        grid=(nb,),
        in_specs=[
            pl.BlockSpec((BM, LATENT), lambda i: (i, 0)),
            pl.BlockSpec((LATENT, CODES), lambda i: (0, 0)),
        ],
        out_specs=pl.BlockSpec((8, BM // 8), lambda i: (i, 0)),
        out_shape=jax.ShapeDtypeStruct((nb * 8, BM // 8), jnp.int32),
    )(flat, embeddings)
    return idx.reshape(-1)


def _sc_gather(table, idx):
    b = idx.shape[0]
    b_per_w = b // NW
    mesh = plsc.VectorSubcoreMesh(core_axis_name="c", subcore_axis_name="s")

    @functools.partial(
        pl.kernel,
        out_type=jax.ShapeDtypeStruct((b, LATENT), jnp.float32),
        mesh=mesh,
        compiler_params=pltpu.CompilerParams(use_tc_tiling_on_sc=False),
        scratch_types=[
            pltpu.VMEM((b_per_w,), jnp.int32),
            pltpu.VMEM((b_per_w, LATENT), jnp.float32),
            pltpu.SemaphoreType.DMA,
        ],
    )
    def gather_kernel(table_hbm, idx_hbm, out_hbm, idx_v, rows_v, sem):
        wid = lax.axis_index("s") * 2 + lax.axis_index("c")
        base = wid * b_per_w
        pltpu.sync_copy(idx_hbm.at[pl.ds(base, b_per_w)], idx_v)
        pltpu.async_copy(table_hbm.at[idx_v], rows_v, sem).wait()
        pltpu.sync_copy(rows_v, out_hbm.at[pl.ds(base, b_per_w)])

    return gather_kernel(table, idx)


def kernel(inputs, embeddings):
    shape = inputs.shape
    flat = inputs.reshape(-1, LATENT)
    idx = _tc_indices(flat, embeddings)
    table = embeddings.T.reshape(CODES, LATENT)
    out = _sc_gather(table, idx)
    return out.reshape(shape)


# speedup vs baseline: 1.9950x; 1.0019x over previous
import functools
import jax
import jax.numpy as jnp
from jax import lax
from jax.experimental import pallas as pl
from jax.experimental.pallas import tpu as pltpu
from jax.experimental.pallas import tpu_sc as plsc

LATENT = 64
CODES = 1024
BM = 1024
NW = 32
CHUNK = 128


def _argmin_block(x_ref, e_ref, idx_ref, et_ref):
    x = x_ref[...]
    e = e_ref[...]
    sim2 = jnp.dot(x, e + e, preferred_element_type=jnp.float32)
    e2 = jnp.sum(e * e, axis=0, keepdims=True)
    x2 = jnp.sum(x * x, axis=1, keepdims=True)
    bestv = (x2 + e2[:, :CHUNK]) - sim2[:, :CHUNK]
    besti0 = lax.broadcasted_iota(jnp.int32, (BM, CHUNK), 1).astype(jnp.float32)
    besti = besti0
    for c in range(1, CODES // CHUNK):
        sl = slice(c * CHUNK, (c + 1) * CHUNK)
        v = (x2 + e2[:, sl]) - sim2[:, sl]
        lt = v < bestv
        bestv = jnp.where(lt, v, bestv)
        besti = jnp.where(lt, besti0 + float(c * CHUNK), besti)
    minval = jnp.min(bestv, axis=1, keepdims=True)
    idxv = jnp.min(jnp.where(bestv == minval, besti, float(CODES)), axis=1)
    idx_ref[...] = idxv.astype(jnp.int32).reshape(8, BM // 8)
    @pl.when(pl.program_id(0) == 0)
    def _():
        rows = lax.broadcasted_iota(jnp.int32, (LATENT, LATENT), 0)
        cols = lax.broadcasted_iota(jnp.int32, (LATENT, LATENT), 1)
        eye = (rows == cols).astype(jnp.float32)
        et_ref[...] = jax.lax.dot_general(
            e, eye, (((0,), (0,)), ((), ())),
            preferred_element_type=jnp.float32)


def _tc_indices(flat, embeddings):
    n = flat.shape[0]
    nb = n // BM
    idx, table = pl.pallas_call(
        _argmin_block,
        grid=(nb,),
        in_specs=[
            pl.BlockSpec((BM, LATENT), lambda i: (i, 0)),
            pl.BlockSpec((LATENT, CODES), lambda i: (0, 0)),
        ],
        out_specs=[
            pl.BlockSpec((8, BM // 8), lambda i: (i, 0)),
            pl.BlockSpec((CODES, LATENT), lambda i: (0, 0)),
        ],
        out_shape=[
            jax.ShapeDtypeStruct((nb * 8, BM // 8), jnp.int32),
            jax.ShapeDtypeStruct((CODES, LATENT), jnp.float32),
        ],
    )(flat, embeddings)
    return idx, table


def _sc_gather(table, idx, b):
    b_per_w = b // NW
    rows_per_w = b_per_w // 128
    mesh = plsc.VectorSubcoreMesh(core_axis_name="c", subcore_axis_name="s")

    @functools.partial(
        pl.kernel,
        out_type=jax.ShapeDtypeStruct((b, LATENT), jnp.float32),
        mesh=mesh,
        compiler_params=pltpu.CompilerParams(use_tc_tiling_on_sc=False),
        scratch_types=[
            pltpu.VMEM((rows_per_w, 128), jnp.int32),
            pltpu.VMEM((b_per_w, LATENT), jnp.float32),
            pltpu.SemaphoreType.DMA,
        ],
    )
    def gather_kernel(table_hbm, idx_hbm, out_hbm, idx_v, rows_v, sem):
        wid = lax.axis_index("s") * 2 + lax.axis_index("c")
        base = wid * b_per_w
        pltpu.sync_copy(idx_hbm.at[pl.ds(wid * rows_per_w, rows_per_w)], idx_v)
        copies = []
        for k in range(rows_per_w):
            copies.append(pltpu.async_copy(
                table_hbm.at[idx_v.at[k]],
                rows_v.at[pl.ds(k * 128, 128)], sem))
        for cp in copies:
            cp.wait()
        pltpu.sync_copy(rows_v, out_hbm.at[pl.ds(base, b_per_w)])

    return gather_kernel(table, idx)


def kernel(inputs, embeddings):
    shape = inputs.shape
    flat = inputs.reshape(-1, LATENT)
    idx, table = _tc_indices(flat, embeddings)
    out = _sc_gather(table, idx, flat.shape[0])
    return out.reshape(shape)
